# baseline (speedup 1.0000x reference)
import functools

import jax
import jax.numpy as jnp
from jax import lax
from jax.experimental import pallas as pl
from jax.experimental.pallas import tpu as pltpu
from jax.experimental.pallas import tpu_sc as plsc

N_NODES = 10000
N_EDGES = 320000
D_FEAT = 128
HIDDEN = 128
BN_EPS = 1e-5

NC = 2
NS = 16
NW = NC * NS
E_PER_W = N_EDGES // NW
CHUNK = 80
NCH = 125
E_PAD_W = NCH * CHUNK
ROWS_PER_TILE = 640
PAD_ROWS = ROWS_PER_TILE * NS

_mesh = plsc.VectorSubcoreMesh(core_axis_name="c", subcore_axis_name="s")


@functools.partial(
    pl.kernel,
    mesh=_mesh,
    out_type=jax.ShapeDtypeStruct((NC * PAD_ROWS, D_FEAT), jnp.float32),
    scratch_types=[
        pltpu.VMEM((E_PAD_W,), jnp.int32),
        pltpu.VMEM((NCH, CHUNK), jnp.int32),
        pltpu.VMEM((CHUNK, D_FEAT), jnp.float32),
        pltpu.VMEM((CHUNK, D_FEAT), jnp.float32),
        pltpu.VMEM_SHARED((PAD_ROWS, D_FEAT), jnp.float32),
        pltpu.SemaphoreType.DMA,
        pltpu.SemaphoreType.DMA,
        pltpu.SemaphoreType.DMA,
        pltpu.SemaphoreType.DMA,
    ],
)
def _agg_kernel(x_hbm, src_hbm, dst_hbm, zeros_hbm, out_hbm,
                src_v, dst_v, rows0_v, rows1_v, acc_sh, sem0, sem1,
                ssem0, ssem1):
    c = lax.axis_index("c")
    s = lax.axis_index("s")
    wid = s * NC + c

    pltpu.sync_copy(zeros_hbm, acc_sh.at[pl.ds(s * ROWS_PER_TILE, ROWS_PER_TILE)])
    pltpu.sync_copy(src_hbm.at[wid], src_v)
    pltpu.sync_copy(dst_hbm.at[wid], dst_v)
    plsc.subcore_barrier()

    pltpu.async_copy(x_hbm.at[src_v.at[pl.ds(0, CHUNK)]], rows0_v, sem0)
    pltpu.async_copy(x_hbm.at[src_v.at[pl.ds(CHUNK, CHUNK)]], rows1_v, sem1)

    def body(i, carry):
        j = 2 * i
        pltpu.make_async_copy(x_hbm.at[pl.ds(0, CHUNK)], rows0_v, sem0).wait()
        pltpu.async_copy(rows0_v, acc_sh.at[dst_v.at[j]], ssem0, add=True)
        pltpu.make_async_copy(x_hbm.at[pl.ds(0, CHUNK)], rows1_v, sem1).wait()
        pltpu.async_copy(rows1_v, acc_sh.at[dst_v.at[j + 1]], ssem1, add=True)
        pltpu.make_async_copy(rows0_v, acc_sh.at[pl.ds(0, CHUNK)], ssem0).wait()
        pltpu.async_copy(x_hbm.at[src_v.at[pl.ds((j + 2) * CHUNK, CHUNK)]],
                         rows0_v, sem0)
        pltpu.make_async_copy(rows1_v, acc_sh.at[pl.ds(0, CHUNK)], ssem1).wait()
        pltpu.async_copy(x_hbm.at[src_v.at[pl.ds((j + 3) * CHUNK, CHUNK)]],
                         rows1_v, sem1)
        return carry

    lax.fori_loop(0, NCH // 2 - 1, body, 0, unroll=False)
    pltpu.make_async_copy(x_hbm.at[pl.ds(0, CHUNK)], rows0_v, sem0).wait()
    pltpu.async_copy(rows0_v, acc_sh.at[dst_v.at[NCH - 3]], ssem0, add=True)
    pltpu.make_async_copy(x_hbm.at[pl.ds(0, CHUNK)], rows1_v, sem1).wait()
    pltpu.async_copy(rows1_v, acc_sh.at[dst_v.at[NCH - 2]], ssem1, add=True)
    pltpu.make_async_copy(rows0_v, acc_sh.at[pl.ds(0, CHUNK)], ssem0).wait()
    pltpu.async_copy(x_hbm.at[src_v.at[pl.ds((NCH - 1) * CHUNK, CHUNK)]],
                     rows0_v, sem0)
    pltpu.make_async_copy(x_hbm.at[pl.ds(0, CHUNK)], rows0_v, sem0).wait()
    pltpu.async_copy(rows0_v, acc_sh.at[dst_v.at[NCH - 1]], ssem0, add=True)
    pltpu.make_async_copy(rows1_v, acc_sh.at[pl.ds(0, CHUNK)], ssem1).wait()
    pltpu.make_async_copy(rows0_v, acc_sh.at[pl.ds(0, CHUNK)], ssem0).wait()
    plsc.subcore_barrier()

    base = c * PAD_ROWS + s * ROWS_PER_TILE
    pltpu.sync_copy(acc_sh.at[pl.ds(s * ROWS_PER_TILE, ROWS_PER_TILE)],
                    out_hbm.at[pl.ds(base, ROWS_PER_TILE)])


def _mlp_body(x_ref, p_ref, w_ref, beta_ref, o_ref):
    h = x_ref[...] + p_ref[0] + p_ref[1]
    y = jnp.dot(h, w_ref[...], preferred_element_type=jnp.float32)
    o_ref[...] = jnp.maximum(y + beta_ref[0:1, :], 0.0)


_BLK = 2000


def kernel(x, edge_index, W, b, bn_weight, bn_bias):
    ei = edge_index.astype(jnp.int32)
    src3 = ei[0].reshape(NW, E_PER_W)
    dst3 = ei[1].reshape(NW, NCH, CHUNK)
    zeros = jnp.zeros((ROWS_PER_TILE, D_FEAT), jnp.float32)

    partials = _agg_kernel(x, src3, dst3, zeros)
    partials = partials.reshape(NC, PAD_ROWS, D_FEAT)

    alpha = bn_weight * (1.0 / jnp.sqrt(1.0 + BN_EPS))
    Wp = (W * alpha[:, None]).T
    beta = jnp.broadcast_to((b * alpha + bn_bias)[None, :], (8, HIDDEN))

    out = pl.pallas_call(
        _mlp_body,
        grid=(N_NODES // _BLK,),
        in_specs=[
            pl.BlockSpec((_BLK, D_FEAT), lambda i: (i, 0)),
            pl.BlockSpec((NC, _BLK, D_FEAT), lambda i: (0, i, 0)),
            pl.BlockSpec((D_FEAT, HIDDEN), lambda i: (0, 0)),
            pl.BlockSpec((8, HIDDEN), lambda i: (0, 0)),
        ],
        out_specs=pl.BlockSpec((_BLK, HIDDEN), lambda i: (i, 0)),
        out_shape=jax.ShapeDtypeStruct((N_NODES, HIDDEN), jnp.float32),
    )(x, partials, Wp, beta)
    return out


# speedup vs baseline: 1.2169x; 1.2169x over previous
import functools

import jax
import jax.numpy as jnp
from jax import lax
from jax.experimental import pallas as pl
from jax.experimental.pallas import tpu as pltpu
from jax.experimental.pallas import tpu_sc as plsc

N_NODES = 10000
N_EDGES = 320000
D_FEAT = 128
HIDDEN = 128
BN_EPS = 1e-5

NC = 2
NS = 16
NW = NC * NS
E_PER_W = N_EDGES // NW
CHUNK = 80
NCH = 125
E_PAD_W = NCH * CHUNK
ROWS_PER_TILE = 640
PAD_ROWS = ROWS_PER_TILE * NS

_mesh = plsc.VectorSubcoreMesh(core_axis_name="c", subcore_axis_name="s")


@functools.partial(
    pl.kernel,
    mesh=_mesh,
    out_type=jax.ShapeDtypeStruct((NC * PAD_ROWS, D_FEAT), jnp.float32),
    scratch_types=[
        pltpu.VMEM((E_PAD_W,), jnp.int32),
        pltpu.VMEM((NCH, CHUNK), jnp.int32),
        pltpu.VMEM((CHUNK, D_FEAT), jnp.float32),
        pltpu.VMEM((CHUNK, D_FEAT), jnp.float32),
        pltpu.VMEM_SHARED((PAD_ROWS, D_FEAT), jnp.float32),
        pltpu.SemaphoreType.DMA,
        pltpu.SemaphoreType.DMA,
    ],
)
def _agg_kernel(x_hbm, src_hbm, dst_hbm, zeros_hbm, out_hbm,
                src_v, dst_v, rows0_v, rows1_v, acc_sh, sem0, sem1):
    c = lax.axis_index("c")
    s = lax.axis_index("s")
    wid = s * NC + c

    pltpu.sync_copy(zeros_hbm, acc_sh.at[pl.ds(s * ROWS_PER_TILE, ROWS_PER_TILE)])
    pltpu.sync_copy(src_hbm.at[wid], src_v)
    pltpu.sync_copy(dst_hbm.at[wid], dst_v)
    plsc.subcore_barrier()

    pltpu.async_copy(x_hbm.at[src_v.at[pl.ds(0, CHUNK)]], rows0_v, sem0)
    pltpu.async_copy(x_hbm.at[src_v.at[pl.ds(CHUNK, CHUNK)]], rows1_v, sem1)

    def body(i, carry):
        j = 2 * i
        pltpu.make_async_copy(x_hbm.at[pl.ds(0, CHUNK)], rows0_v, sem0).wait()
        pltpu.sync_copy(rows0_v, acc_sh.at[dst_v.at[j]], add=True)
        pltpu.async_copy(x_hbm.at[src_v.at[pl.ds((j + 2) * CHUNK, CHUNK)]],
                         rows0_v, sem0)
        pltpu.make_async_copy(x_hbm.at[pl.ds(0, CHUNK)], rows1_v, sem1).wait()
        pltpu.sync_copy(rows1_v, acc_sh.at[dst_v.at[j + 1]], add=True)
        pltpu.async_copy(x_hbm.at[src_v.at[pl.ds((j + 3) * CHUNK, CHUNK)]],
                         rows1_v, sem1)
        return carry

    lax.fori_loop(0, NCH // 2 - 1, body, 0, unroll=False)
    pltpu.make_async_copy(x_hbm.at[pl.ds(0, CHUNK)], rows0_v, sem0).wait()
    pltpu.sync_copy(rows0_v, acc_sh.at[dst_v.at[NCH - 3]], add=True)
    pltpu.async_copy(x_hbm.at[src_v.at[pl.ds((NCH - 1) * CHUNK, CHUNK)]],
                     rows0_v, sem0)
    pltpu.make_async_copy(x_hbm.at[pl.ds(0, CHUNK)], rows1_v, sem1).wait()
    pltpu.sync_copy(rows1_v, acc_sh.at[dst_v.at[NCH - 2]], add=True)
    pltpu.make_async_copy(x_hbm.at[pl.ds(0, CHUNK)], rows0_v, sem0).wait()
    pltpu.sync_copy(rows0_v, acc_sh.at[dst_v.at[NCH - 1]], add=True)
    plsc.subcore_barrier()

    base = c * PAD_ROWS + s * ROWS_PER_TILE
    pltpu.sync_copy(acc_sh.at[pl.ds(s * ROWS_PER_TILE, ROWS_PER_TILE)],
                    out_hbm.at[pl.ds(base, ROWS_PER_TILE)])


def _mlp_body(x_ref, p_ref, w_ref, beta_ref, o_ref):
    h = x_ref[...] + p_ref[0] + p_ref[1]
    y = jnp.dot(h, w_ref[...], preferred_element_type=jnp.float32)
    o_ref[...] = jnp.maximum(y + beta_ref[0:1, :], 0.0)


_BLK = 2000


def kernel(x, edge_index, W, b, bn_weight, bn_bias):
    ei = edge_index.astype(jnp.int32)
    src3 = ei[0].reshape(NW, E_PER_W)
    dst3 = ei[1].reshape(NW, NCH, CHUNK)
    zeros = jnp.zeros((ROWS_PER_TILE, D_FEAT), jnp.float32)

    partials = _agg_kernel(x, src3, dst3, zeros)
    partials = partials.reshape(NC, PAD_ROWS, D_FEAT)

    alpha = bn_weight * (1.0 / jnp.sqrt(1.0 + BN_EPS))
    Wp = (W * alpha[:, None]).T
    beta = jnp.broadcast_to((b * alpha + bn_bias)[None, :], (8, HIDDEN))

    out = pl.pallas_call(
        _mlp_body,
        grid=(N_NODES // _BLK,),
        in_specs=[
            pl.BlockSpec((_BLK, D_FEAT), lambda i: (i, 0)),
            pl.BlockSpec((NC, _BLK, D_FEAT), lambda i: (0, i, 0)),
            pl.BlockSpec((D_FEAT, HIDDEN), lambda i: (0, 0)),
            pl.BlockSpec((8, HIDDEN), lambda i: (0, 0)),
        ],
        out_specs=pl.BlockSpec((_BLK, HIDDEN), lambda i: (i, 0)),
        out_shape=jax.ShapeDtypeStruct((N_NODES, HIDDEN), jnp.float32),
    )(x, partials, Wp, beta)
    return out
